# Initial kernel scaffold; baseline (speedup 1.0000x reference)
import jax
import jax.numpy as jnp
from jax.experimental import pallas as pl


def kernel(x, edge_index, W1, attn_l1, attn_r1, W2, attn_l2, attn_r2):
    raise NotImplementedError("write your pallas kernel here")


# speedup vs baseline: 37.2385x; 37.2385x over previous
import functools

import jax
import jax.numpy as jnp
from jax import lax
from jax.experimental import pallas as pl
from jax.experimental.pallas import tpu as pltpu
from jax.experimental.pallas import tpu_sc as plsc

_NC = 2
_NS = 16
_L = 16


def _prep1_body(x_ref, w_ref, al_ref, ar_ref, g_ref, er_ref, cm_ref):
    feat = jnp.dot(x_ref[...], w_ref[...], preferred_element_type=jnp.float32)
    el = jnp.dot(feat, al_ref[...], preferred_element_type=jnp.float32)
    er = jnp.dot(feat, ar_ref[...], preferred_element_type=jnp.float32)
    c = jnp.max(el, axis=0, keepdims=True) + jnp.max(er, axis=0, keepdims=True)
    cm = jnp.where(c > 0, c, 0.2 * c)
    n, h = el.shape
    fw = feat.shape[1] // _NC
    hh = h // _NC
    halves_g = []
    halves_e = []
    halves_c = []
    for cc in range(_NC):
        halves_g.append(jnp.concatenate(
            [feat[:, cc * fw:(cc + 1) * fw],
             el[:, cc * hh:(cc + 1) * hh],
             jnp.zeros((n, _L - hh), jnp.float32)], axis=1))
        halves_e.append(jnp.concatenate(
            [er[:, cc * hh:(cc + 1) * hh],
             jnp.full((n, _L - hh), -60.0, jnp.float32)], axis=1))
        halves_c.append(jnp.concatenate(
            [cm[:, cc * hh:(cc + 1) * hh],
             jnp.zeros((1, _L - hh), jnp.float32)], axis=1))
    g_ref[...] = jnp.concatenate(halves_g, axis=0)
    er_ref[...] = jnp.concatenate(halves_e, axis=0)
    cm_ref[...] = jnp.concatenate(halves_c, axis=0)


def _mid_body(n_nodes, o1_ref, w2_ref, al2_ref, ar2_ref, s1_ref,
              g2_ref, er2_ref, cm2_ref):
    d = w2_ref.shape[0]
    h = s1_ref.shape[0]
    fw = d // _NC
    hh = h // _NC
    a0 = o1_ref[0]
    a1 = o1_ref[1]
    num = jnp.concatenate([a0[:n_nodes, :fw], a1[:n_nodes, :fw]], axis=1)
    den = jnp.concatenate([a0[:n_nodes, fw:fw + hh],
                           a1[:n_nodes, fw:fw + hh]], axis=1)
    den_w = jnp.dot(den, s1_ref[...], preferred_element_type=jnp.float32)
    q = num / (den_w + 1e-9)
    h1 = jnp.where(q > 0, q, jnp.exp(q) - 1.0)
    feat2 = jnp.dot(h1, w2_ref[...], preferred_element_type=jnp.float32)
    el2 = jnp.dot(feat2, al2_ref[...], preferred_element_type=jnp.float32)
    er2 = jnp.dot(feat2, ar2_ref[...], preferred_element_type=jnp.float32)
    c2 = jnp.max(el2, axis=0, keepdims=True) + jnp.max(er2, axis=0,
                                                       keepdims=True)
    cm2 = jnp.where(c2 > 0, c2, 0.2 * c2)
    h2 = al2_ref.shape[1]
    g2_ref[...] = jnp.concatenate(
        [feat2, el2, jnp.zeros((n_nodes, _L - h2), jnp.float32)], axis=1)
    er2_ref[...] = jnp.concatenate(
        [er2, jnp.full((n_nodes, _L - h2), -60.0, jnp.float32)], axis=1)
    cm2_row = jnp.concatenate(
        [cm2, jnp.zeros((1, _L - h2), jnp.float32)], axis=1)
    cm2_ref[...] = jnp.concatenate([cm2_row] * _NC, axis=0)


def _fin_body(o2_ref, out_ref):
    a = o2_ref[0] + o2_ref[1]
    n, d = out_ref.shape
    num = a[:n, :d]
    den = a[:n, d:d + 1]
    out_ref[...] = num / (den + 1e-9)


def _edge_pass(gtab, ertab, cmtab, src, dst, n_nodes, hh, dim, split_heads):
    e_total = src.shape[0]
    nworie = _NS if split_heads else _NC * _NS
    ew = e_total // nworie
    nchunk = ew // 128
    tail = ew - nchunk * 128
    np_rows = -(-n_nodes // (_NS * 8)) * (_NS * 8)
    rpt = np_rows // _NS
    gw = dim * hh + _L
    fd = dim * hh
    zrows = 8
    assert rpt % zrows == 0 and tail % _L == 0

    mesh = plsc.VectorSubcoreMesh(core_axis_name="c", subcore_axis_name="s",
                                  num_cores=_NC, num_subcores=_NS)

    def body(g_hbm, er_hbm, cm_hbm, src_hbm, dst_hbm, o_hbm,
             srcv, dstv, soff, gbuf, erbuf, obuf,
             srcv2, dstv2, soff2, gbuf2, erbuf2, obuf2, zbuf, cmbuf, ashr):
        cc = lax.axis_index("c")
        ss = lax.axis_index("s")
        row0 = ss * rpt
        rowoff = cc * n_nodes
        pltpu.sync_copy(cm_hbm.at[cc], cmbuf)

        zero = jnp.zeros((_L,), jnp.float32)
        for r in range(zrows):
            for j in range(gw // _L):
                zbuf[r, pl.ds(_L * j, _L)] = zero
        for b in range(rpt // zrows):
            pltpu.sync_copy(zbuf, ashr.at[pl.ds(row0 + zrows * b, zrows)])
        plsc.subcore_barrier()

        ew0 = (ss if split_heads else ss * _NC + cc) * ew

        def do_chunk(off, sv, dv, so, gb, eb, ob, csz):
            off = pl.multiple_of(off, 8)
            pltpu.sync_copy(src_hbm.at[pl.ds(off, csz)], sv)
            pltpu.sync_copy(dst_hbm.at[pl.ds(off, csz)], dv)
            if split_heads:
                for g in range(csz // _L):
                    so[pl.ds(_L * g, _L)] = sv[pl.ds(_L * g, _L)] + rowoff
                pltpu.sync_copy(g_hbm.at[so], gb)
                for g in range(csz // _L):
                    so[pl.ds(_L * g, _L)] = dv[pl.ds(_L * g, _L)] + rowoff
                pltpu.sync_copy(er_hbm.at[so], eb)
            else:
                pltpu.sync_copy(g_hbm.at[sv], gb)
                pltpu.sync_copy(er_hbm.at[dv], eb)

            cmv = cmbuf[:]

            def ebody(i, carry):
                elv = gb[i, pl.ds(fd, _L)]
                erv = eb[i, :]
                s = elv + erv
                e = jnp.where(s > 0, s, 0.2 * s) - cmv
                ee = jnp.exp(e)
                ob[i, pl.ds(fd, _L)] = ee
                for h in range(hh):
                    ob[i, pl.ds(dim * h, _L)] = (
                        gb[i, pl.ds(dim * h, _L)] * ee[h])
                return carry

            lax.fori_loop(0, csz, ebody, 0)
            pltpu.sync_copy(ob, ashr.at[dv], add=True)

        def chunk_loop(k, carry):
            do_chunk(ew0 + 128 * k, srcv, dstv, soff, gbuf, erbuf, obuf, 128)
            return carry

        lax.fori_loop(0, nchunk, chunk_loop, 0)
        if tail:
            do_chunk(ew0 + 128 * nchunk, srcv2, dstv2, soff2,
                     gbuf2, erbuf2, obuf2, tail)
        plsc.subcore_barrier()
        pltpu.sync_copy(ashr.at[pl.ds(row0, rpt)],
                        o_hbm.at[cc, pl.ds(row0, rpt)])

    scratch = [
        pltpu.VMEM((128,), jnp.int32), pltpu.VMEM((128,), jnp.int32),
        pltpu.VMEM((128,), jnp.int32),
        pltpu.VMEM((128, gw), jnp.float32),
        pltpu.VMEM((128, _L), jnp.float32),
        pltpu.VMEM((128, gw), jnp.float32),
        pltpu.VMEM((tail or 8,), jnp.int32), pltpu.VMEM((tail or 8,), jnp.int32),
        pltpu.VMEM((tail or 8,), jnp.int32),
        pltpu.VMEM((tail or 8, gw), jnp.float32),
        pltpu.VMEM((tail or 8, _L), jnp.float32),
        pltpu.VMEM((tail or 8, gw), jnp.float32),
        pltpu.VMEM((zrows, gw), jnp.float32),
        pltpu.VMEM((_L,), jnp.float32),
        pltpu.VMEM_SHARED((np_rows, gw), jnp.float32),
    ]
    out_type = jax.ShapeDtypeStruct((_NC, np_rows, gw), jnp.float32)
    return pl.kernel(
        body, out_type=out_type, mesh=mesh, scratch_types=scratch,
        compiler_params=pltpu.CompilerParams(use_tc_tiling_on_sc=False),
    )(gtab, ertab, cmtab, src, dst)


def kernel(x, edge_index, W1, attn_l1, attn_r1, W2, attn_l2, attn_r2):
    n, _ = x.shape
    h1, hid = attn_l1.shape
    h2, out_d = attn_l2.shape
    src = edge_index[0].astype(jnp.int32)
    dst = edge_index[1].astype(jnp.int32)

    al1 = (jnp.eye(h1, dtype=jnp.float32)[:, None, :]
           * attn_l1[:, :, None]).reshape(h1 * hid, h1)
    ar1 = (jnp.eye(h1, dtype=jnp.float32)[:, None, :]
           * attn_r1[:, :, None]).reshape(h1 * hid, h1)
    al2 = (jnp.eye(h2, dtype=jnp.float32)[:, None, :]
           * attn_l2[:, :, None]).reshape(h2 * out_d, h2)
    ar2 = (jnp.eye(h2, dtype=jnp.float32)[:, None, :]
           * attn_r2[:, :, None]).reshape(h2 * out_d, h2)
    s1 = jnp.repeat(jnp.eye(h1, dtype=jnp.float32), hid, axis=1)

    gw1 = (h1 // _NC) * hid + _L
    g1, er1, cm1 = pl.pallas_call(
        _prep1_body,
        out_shape=[jax.ShapeDtypeStruct((_NC * n, gw1), jnp.float32),
                   jax.ShapeDtypeStruct((_NC * n, _L), jnp.float32),
                   jax.ShapeDtypeStruct((_NC, _L), jnp.float32)],
    )(x, W1, al1, ar1)

    o1 = _edge_pass(g1, er1, cm1, src, dst, n, h1 // _NC, hid, True)

    g2, er2, cm2 = pl.pallas_call(
        functools.partial(_mid_body, n),
        out_shape=[jax.ShapeDtypeStruct((n, h2 * out_d + _L), jnp.float32),
                   jax.ShapeDtypeStruct((n, _L), jnp.float32),
                   jax.ShapeDtypeStruct((_NC, _L), jnp.float32)],
    )(o1, W2, al2, ar2, s1)

    o2 = _edge_pass(g2, er2, cm2, src, dst, n, h2, out_d, False)

    out = pl.pallas_call(
        _fin_body,
        out_shape=jax.ShapeDtypeStruct((n, out_d), jnp.float32),
    )(o2)
    return out
